# baseline (device time: 70409 ns/iter reference)
import jax
import jax.numpy as jnp
from jax import lax
from jax.experimental import pallas as pl
from jax.experimental.pallas import tpu as pltpu

N_DEV = 8
B = 2
SQ = 256
D = 768
H = 8
DH = 64
SKV = 512
ROWS = B * SQ
CHUNK = ROWS // N_DEV
SCALE = 0.125


def kernel(x, Wq, Wo, K_ext, V_ext):
    x2 = x.reshape(ROWS, D)
    k2 = jnp.transpose(K_ext, (0, 2, 1, 3)).reshape(B * H, SKV, DH)
    v2 = jnp.transpose(V_ext, (0, 2, 1, 3)).reshape(B * H, SKV, DH)

    def body(x_ref, wq_ref, wo_ref, k_ref, v_ref, out_ref,
             q_ref, attn_ref, acc_ref, comm_ref,
             rs_send, rs_recv, ag_send, ag_recv):
        my = lax.axis_index("i")
        left = jnp.mod(my - 1, N_DEV)
        right = jnp.mod(my + 1, N_DEV)

        barrier = pltpu.get_barrier_semaphore()
        for nbr in (left, right):
            pl.semaphore_signal(barrier, inc=1, device_id=(nbr,),
                                device_id_type=pl.DeviceIdType.MESH)
        pl.semaphore_wait(barrier, 2)

        q_ref[...] = jnp.dot(x_ref[...], wq_ref[...],
                             preferred_element_type=jnp.float32)
        for b in range(B):
            for h in range(H):
                i = b * H + h
                q = q_ref[b * SQ:(b + 1) * SQ, h * DH:(h + 1) * DH]
                s = lax.dot_general(
                    q, k_ref[i], (((1,), (1,)), ((), ())),
                    preferred_element_type=jnp.float32) * SCALE
                m = jnp.max(s, axis=1, keepdims=True)
                p = jnp.exp(s - m)
                l = jnp.sum(p, axis=1, keepdims=True)
                o = jnp.dot(p, v_ref[i],
                            preferred_element_type=jnp.float32) / l
                attn_ref[b * SQ:(b + 1) * SQ, h * DH:(h + 1) * DH] = o

        acc_ref[...] = jnp.dot(attn_ref[...], wo_ref[...],
                               preferred_element_type=jnp.float32)

        for s in range(N_DEV - 1):
            send_c = jnp.mod(my - s, N_DEV)
            rdma = pltpu.make_async_remote_copy(
                src_ref=acc_ref.at[pl.ds(send_c * CHUNK, CHUNK), :],
                dst_ref=comm_ref.at[s],
                send_sem=rs_send.at[s],
                recv_sem=rs_recv.at[s],
                device_id=(right,),
                device_id_type=pl.DeviceIdType.MESH,
            )
            rdma.start()
            rdma.wait()
            recv_c = jnp.mod(my - s - 1, N_DEV)
            acc_ref[pl.ds(recv_c * CHUNK, CHUNK), :] = (
                acc_ref[pl.ds(recv_c * CHUNK, CHUNK), :] + comm_ref[s])

        own = jnp.mod(my + 1, N_DEV)
        out_ref[pl.ds(own * CHUNK, CHUNK), :] = acc_ref[
            pl.ds(own * CHUNK, CHUNK), :]
        for s in range(N_DEV - 1):
            q_c = jnp.mod(my + 1 - s, N_DEV)
            rdma = pltpu.make_async_remote_copy(
                src_ref=out_ref.at[pl.ds(q_c * CHUNK, CHUNK), :],
                dst_ref=out_ref.at[pl.ds(q_c * CHUNK, CHUNK), :],
                send_sem=ag_send.at[s],
                recv_sem=ag_recv.at[s],
                device_id=(right,),
                device_id_type=pl.DeviceIdType.MESH,
            )
            rdma.start()
            rdma.wait()

    out = pl.pallas_call(
        body,
        out_shape=jax.ShapeDtypeStruct((ROWS, D), jnp.float32),
        in_specs=[pl.BlockSpec(memory_space=pltpu.VMEM)] * 5,
        out_specs=pl.BlockSpec(memory_space=pltpu.VMEM),
        scratch_shapes=[
            pltpu.VMEM((ROWS, H * DH), jnp.float32),
            pltpu.VMEM((ROWS, H * DH), jnp.float32),
            pltpu.VMEM((ROWS, D), jnp.float32),
            pltpu.VMEM((N_DEV - 1, CHUNK, D), jnp.float32),
            pltpu.SemaphoreType.DMA((N_DEV - 1,)),
            pltpu.SemaphoreType.DMA((N_DEV - 1,)),
            pltpu.SemaphoreType.DMA((N_DEV - 1,)),
            pltpu.SemaphoreType.DMA((N_DEV - 1,)),
        ],
        compiler_params=pltpu.CompilerParams(collective_id=0),
    )(x2, Wq, Wo, k2, v2)
    return out.reshape(B, SQ, D)


# device time: 28388 ns/iter; 2.4802x vs baseline; 2.4802x over previous
import jax
import jax.numpy as jnp
from jax import lax
from jax.experimental import pallas as pl
from jax.experimental.pallas import tpu as pltpu

N_DEV = 8
B = 2
SQ = 256
D = 768
H = 8
DH = 64
SKV = 512
ROWS = B * SQ
CHUNK = ROWS // N_DEV
SCALE = 0.125


def kernel(x, Wq, Wo, K_ext, V_ext):
    x2 = x.reshape(ROWS, D)
    k2 = jnp.transpose(K_ext, (0, 2, 1, 3)).reshape(B * H, SKV, DH)
    v2 = jnp.transpose(V_ext, (0, 2, 1, 3)).reshape(B * H, SKV, DH)

    def body(x_ref, wq_ref, wo_ref, k_ref, v_ref, out_ref,
             q_ref, attn_ref, own_ref, sendb_ref, rs_buf, redb_ref, ag_buf,
             rs_send, rs_recv, ag_send, ag_recv):
        my = lax.axis_index("i")

        barrier = pltpu.get_barrier_semaphore()
        for d in range(N_DEV):
            @pl.when(d != my)
            def _():
                pl.semaphore_signal(barrier, inc=1, device_id=(d,),
                                    device_id_type=pl.DeviceIdType.MESH)
        pl.semaphore_wait(barrier, N_DEV - 1)

        q_ref[...] = jnp.dot(x_ref[...], wq_ref[...],
                             preferred_element_type=jnp.float32)
        for b in range(B):
            for h in range(H):
                i = b * H + h
                q = q_ref[b * SQ:(b + 1) * SQ, h * DH:(h + 1) * DH]
                s = lax.dot_general(
                    q, k_ref[i], (((1,), (1,)), ((), ())),
                    preferred_element_type=jnp.float32) * SCALE
                m = jnp.max(s, axis=1, keepdims=True)
                p = jnp.exp(s - m)
                l = jnp.sum(p, axis=1, keepdims=True)
                o = jnp.dot(p, v_ref[i],
                            preferred_element_type=jnp.float32) / l
                attn_ref[b * SQ:(b + 1) * SQ, h * DH:(h + 1) * DH] = o

        rs_rdmas = []
        for t in range(N_DEV):
            c = jnp.mod(my + 1 + t, N_DEV)
            blk = jnp.dot(attn_ref[pl.ds(c * CHUNK, CHUNK), :], wo_ref[...],
                          preferred_element_type=jnp.float32)
            if t < N_DEV - 1:
                sendb_ref[t] = blk.astype(jnp.bfloat16)
                slot = jnp.mod(my - c - 1, N_DEV)
                rdma = pltpu.make_async_remote_copy(
                    src_ref=sendb_ref.at[t],
                    dst_ref=rs_buf.at[slot],
                    send_sem=rs_send.at[t],
                    recv_sem=rs_recv.at[slot],
                    device_id=(c,),
                    device_id_type=pl.DeviceIdType.MESH,
                )
                rdma.start()
                rs_rdmas.append(rdma)
            else:
                own_ref[...] = blk

        for s in range(N_DEV - 1):
            pltpu.make_async_remote_copy(
                src_ref=rs_buf.at[s], dst_ref=rs_buf.at[s],
                send_sem=rs_send.at[0], recv_sem=rs_recv.at[s],
                device_id=(my,), device_id_type=pl.DeviceIdType.MESH,
            ).wait_recv()
        red = own_ref[...]
        for s in range(N_DEV - 1):
            red = red + rs_buf[s].astype(jnp.float32)
        out_ref[pl.ds(my * CHUNK, CHUNK), :] = red
        redb_ref[...] = red.astype(jnp.bfloat16)

        ag_rdmas = []
        t = 0
        for d in range(1, N_DEV):
            dst = jnp.mod(my + d, N_DEV)
            rdma = pltpu.make_async_remote_copy(
                src_ref=redb_ref,
                dst_ref=ag_buf.at[my],
                send_sem=ag_send.at[t],
                recv_sem=ag_recv.at[my],
                device_id=(dst,),
                device_id_type=pl.DeviceIdType.MESH,
            )
            rdma.start()
            ag_rdmas.append(rdma)
            t += 1

        for c in range(N_DEV):
            @pl.when(c != my)
            def _():
                pltpu.make_async_remote_copy(
                    src_ref=redb_ref, dst_ref=ag_buf.at[c],
                    send_sem=ag_send.at[0], recv_sem=ag_recv.at[c],
                    device_id=(my,), device_id_type=pl.DeviceIdType.MESH,
                ).wait_recv()
                out_ref[pl.ds(c * CHUNK, CHUNK), :] = (
                    ag_buf[c].astype(jnp.float32))

        for rdma in rs_rdmas + ag_rdmas:
            rdma.wait_send()

    out = pl.pallas_call(
        body,
        out_shape=jax.ShapeDtypeStruct((ROWS, D), jnp.float32),
        in_specs=[pl.BlockSpec(memory_space=pltpu.VMEM)] * 5,
        out_specs=pl.BlockSpec(memory_space=pltpu.VMEM),
        scratch_shapes=[
            pltpu.VMEM((ROWS, H * DH), jnp.float32),
            pltpu.VMEM((ROWS, H * DH), jnp.float32),
            pltpu.VMEM((CHUNK, D), jnp.float32),
            pltpu.VMEM((N_DEV - 1, CHUNK, D), jnp.bfloat16),
            pltpu.VMEM((N_DEV - 1, CHUNK, D), jnp.bfloat16),
            pltpu.VMEM((CHUNK, D), jnp.bfloat16),
            pltpu.VMEM((N_DEV, CHUNK, D), jnp.bfloat16),
            pltpu.SemaphoreType.DMA((N_DEV - 1,)),
            pltpu.SemaphoreType.DMA((N_DEV - 1,)),
            pltpu.SemaphoreType.DMA((N_DEV - 1,)),
            pltpu.SemaphoreType.DMA((N_DEV,)),
        ],
        compiler_params=pltpu.CompilerParams(collective_id=0),
    )(x2, Wq, Wo, k2, v2)
    return out.reshape(B, SQ, D)
